# baseline (speedup 1.0000x reference)
import numpy as np
import jax
import jax.numpy as jnp
from jax import lax
from jax.experimental import pallas as pl
from jax.experimental.pallas import tpu as pltpu
from jax.experimental.pallas import tpu_sc as plsc

_D = 64
_S = 20
_M = 50
_B = 1024

_NC, _NS = 2, 16
_NW = _NC * _NS
_BPW = _B // _NW
_CH = 25
_NCH = _BPW * (_M // _CH)
_G = _CH * _S
_GPART = [(0, 6), (6, 6), (12, 6), (18, 7)]


def _pos_enc():
    j = np.arange(1, _S + 1, dtype=np.float32)[:, None]
    k = np.arange(1, _D + 1, dtype=np.float32)[None, :]
    return jnp.asarray((1.0 - j / _S) - (k / _D) * (1.0 - 2.0 * j / _S))


def _body(x_hbm, table_hbm, te_hbm, w_hbm, out_hbm,
          idx0, idx1, rows0, rows1, out_v, te_v, w_v, sem0, sem1):
    wid = lax.axis_index("s") * _NC + lax.axis_index("c")
    b_base = wid * _BPW
    pltpu.sync_copy(w_hbm, w_v)
    pltpu.sync_copy(te_hbm, te_v)
    idx_b, rows_b, sem_b = (idx0, idx1), (rows0, rows1), (sem0, sem1)

    def chunk_pos(c):
        return b_base + lax.shift_right_logical(c, 1), (c & 1) * _CH

    def fire(c, b):
        bi, m0 = chunk_pos(c)
        pltpu.sync_copy(x_hbm.at[bi, pl.ds(m0, _CH)], idx_b[b])
        for r0 in range(_CH):
            pltpu.make_async_copy(
                table_hbm.at[idx_b[b].at[r0]],
                rows_b[b].at[pl.ds(r0 * _S, _S)],
                sem_b[b],
            ).start()

    def drain(b):
        pltpu.make_async_copy(
            table_hbm.at[pl.ds(0, _G)], rows_b[b], sem_b[b]
        ).wait()

    def compute(c, b):
        rows_v = rows_b[b]
        bi, m0 = chunk_pos(c)

        def row(r, carry2):
            r0 = r * _S
            for jj in range(_D // 16):
                dsl = pl.ds(jj * 16, 16)
                acc = te_v[m0 + r, dsl]
                for k in range(_S):
                    acc = acc + rows_v[r0 + k, dsl] * w_v[k, dsl]
                out_v[r, dsl] = acc
            return carry2

        lax.fori_loop(0, _CH, row, 0)
        pltpu.sync_copy(out_v, out_hbm.at[bi, pl.ds(m0, _CH)])

    fire(0, 0)
    fire(1, 1)

    def pair(i, carry):
        for b in range(2):
            c = i * 2 + b
            drain(b)
            compute(c, b)

            @pl.when(c + 2 < _NCH)
            def _():
                fire(c + 2, b)
        return carry

    lax.fori_loop(0, _NCH // 2, pair, 0)


def kernel(x, embedding, temporal_embedding):
    mesh = plsc.VectorSubcoreMesh(core_axis_name="c", subcore_axis_name="s")
    out = pl.kernel(
        _body,
        mesh=mesh,
        compiler_params=pltpu.CompilerParams(use_tc_tiling_on_sc=False),
        out_type=jax.ShapeDtypeStruct((_B, _M, _D), jnp.float32),
        scratch_types=[
            pltpu.VMEM((_CH, _S), jnp.int32),
            pltpu.VMEM((_CH, _S), jnp.int32),
            pltpu.VMEM((_G, _D), jnp.float32),
            pltpu.VMEM((_G, _D), jnp.float32),
            pltpu.VMEM((_CH, _D), jnp.float32),
            pltpu.VMEM((_M, _D), jnp.float32),
            pltpu.VMEM((_S, _D), jnp.float32),
            pltpu.SemaphoreType.DMA,
            pltpu.SemaphoreType.DMA,
        ],
    )(x, embedding, temporal_embedding, _pos_enc())
    return out


# speedup vs baseline: 1.1209x; 1.1209x over previous
import numpy as np
import jax
import jax.numpy as jnp
from jax import lax
from jax.experimental import pallas as pl
from jax.experimental.pallas import tpu as pltpu
from jax.experimental.pallas import tpu_sc as plsc

_D = 64
_S = 20
_M = 50
_B = 1024

_NC, _NS = 2, 16
_NW = _NC * _NS
_ROWS = _B * _M
_RPW = _ROWS // _NW
_CH = 80
_NCH = _RPW // _CH
_G = _CH * _S
_GPART = [(j * 128, 128) for j in range(12)] + [(1536, 64)]

_PERM = np.arange(_D).reshape(2, 2, 16).transpose(0, 2, 1).reshape(-1)


def _pos_enc():
    j = np.arange(1, _S + 1, dtype=np.float32)[:, None]
    k = np.arange(1, _D + 1, dtype=np.float32)[None, :]
    return jnp.asarray((1.0 - j / _S) - (k / _D) * (1.0 - 2.0 * j / _S))


def _body(x_hbm, table_hbm, te_hbm, w_hbm, out_hbm,
          idx0, idx1, rows0, rows1, out_v, te_v, w_v, sem0, sem1):
    wid = lax.axis_index("s") * _NC + lax.axis_index("c")
    base = wid * _RPW
    pltpu.sync_copy(w_hbm, w_v)
    pltpu.sync_copy(te_hbm, te_v)
    idx_b, rows_b, sem_b = (idx0, idx1), (rows0, rows1), (sem0, sem1)

    def fire(c, b):
        pltpu.sync_copy(x_hbm.at[pl.ds((base + c * _CH) * _S, _G)], idx_b[b])
        for off, sz in _GPART:
            pltpu.make_async_copy(
                table_hbm.at[idx_b[b].at[pl.ds(off, sz)]],
                rows_b[b].at[pl.ds(off, sz)],
                sem_b[b],
            ).start()

    def drain(b):
        pltpu.make_async_copy(
            table_hbm.at[pl.ds(0, _G)], rows_b[b], sem_b[b]
        ).wait()

    def compute(c, b):
        rows_v = rows_b[b]
        m0 = lax.rem(c * _CH, _M)

        def row(r, carry2):
            r0 = r * _S
            mr = lax.rem(m0 + r, _M)
            for jj in range(_D // 32):
                la, lb = pl.ds(jj * 32, 16), pl.ds(jj * 32 + 16, 16)
                acc_a = te_v[mr, la]
                acc_b = te_v[mr, lb]
                for k in range(_S):
                    ab = rows_v[r0 + k, pl.ds(jj * 32, 32)]
                    a, b2 = plsc.unpack(ab, format=plsc.PackFormat.INTERLEAVED)
                    acc_a = acc_a + a * w_v[k, la]
                    acc_b = acc_b + b2 * w_v[k, lb]
                out_v[r, la] = acc_a
                out_v[r, lb] = acc_b
            return carry2

        lax.fori_loop(0, _CH, row, 0)
        pltpu.sync_copy(out_v, out_hbm.at[pl.ds(base + c * _CH, _CH)])

    fire(0, 0)
    fire(1, 1)

    def pair(i, carry):
        for b in range(2):
            c = i * 2 + b
            drain(b)
            compute(c, b)

            @pl.when(c + 2 < _NCH)
            def _():
                fire(c + 2, b)
        return carry

    lax.fori_loop(0, _NCH // 2, pair, 0)


def kernel(x, embedding, temporal_embedding):
    table = embedding.astype(jnp.bfloat16)[:, _PERM]
    mesh = plsc.VectorSubcoreMesh(core_axis_name="c", subcore_axis_name="s")
    out = pl.kernel(
        _body,
        mesh=mesh,
        compiler_params=pltpu.CompilerParams(
            use_tc_tiling_on_sc=False, needs_layout_passes=False
        ),
        out_type=jax.ShapeDtypeStruct((_ROWS, _D), jnp.float32),
        scratch_types=[
            pltpu.VMEM((_G,), jnp.int32),
            pltpu.VMEM((_G,), jnp.int32),
            pltpu.VMEM((_G, _D), jnp.bfloat16),
            pltpu.VMEM((_G, _D), jnp.bfloat16),
            pltpu.VMEM((_CH, _D), jnp.float32),
            pltpu.VMEM((_M, _D), jnp.float32),
            pltpu.VMEM((_S, _D), jnp.float32),
            pltpu.SemaphoreType.DMA,
            pltpu.SemaphoreType.DMA,
        ],
    )(x.reshape(-1), table, temporal_embedding, _pos_enc())
    return out.reshape(_B, _M, _D)
